# baseline (speedup 1.0000x reference)
import functools

import jax
import jax.numpy as jnp
from jax.experimental import pallas as pl
from jax.experimental.pallas import tpu as pltpu

_B, _H, _W, _C = 16, 512, 512, 3
_P = 8
_OUT = 64
_WIN_Y = 136
_WIN_X = 256
_EPS = 1000.0 * float(jnp.finfo(jnp.float32).eps)


def _weights(length, off, win, *, transposed):
    lf = length.astype(jnp.float32)
    inv_scale = lf * (1.0 / _OUT)
    ks = jnp.maximum(inv_scale, 1.0)
    if transposed:
        shape = (_OUT, win)
        s_dim, i_dim, red_axis = 0, 1, 1
    else:
        shape = (win, _OUT)
        s_dim, i_dim, red_axis = 1, 0, 0
    s = jax.lax.broadcasted_iota(jnp.int32, shape, s_dim).astype(jnp.float32)
    i = jax.lax.broadcasted_iota(jnp.int32, shape, i_dim).astype(jnp.float32)
    sample = (s + 0.5) * inv_scale - 0.5
    r = i - off.astype(jnp.float32)
    x = jnp.abs(sample - r) * (1.0 / ks)
    w = jnp.maximum(0.0, 1.0 - x)
    w = w * ((r >= 0.0) & (r < lf)).astype(jnp.float32)
    total = jnp.sum(w, axis=red_axis, keepdims=True)
    return w * (1.0 / total)


def _body(g_ref, boxes_ref, pg_ref, out_ref):
    b = pl.program_id(0)

    @pl.when(b == 0)
    def _init():
        out_ref[...] = jnp.zeros_like(out_ref)

    accs = [jnp.zeros((_OUT, _OUT), jnp.float32) for _ in range(_C)]
    for p in range(_P):
        ymin = boxes_ref[b, p, 0]
        xmin = boxes_ref[b, p, 1]
        ph = boxes_ref[b, p, 2]
        pw = boxes_ref[b, p, 3]
        ys = (jnp.minimum(ymin, _H - _WIN_Y + 5) // 8) * 8
        xs = (jnp.minimum(xmin, _W - _WIN_X + 1) // 128) * 128
        wyt = _weights(ph, ymin - ys, _WIN_Y, transposed=True).astype(
            jnp.bfloat16)
        wx = _weights(pw, xmin - xs, _WIN_X, transposed=False).astype(
            jnp.bfloat16)
        for c in range(_C):
            crop = g_ref[0, c, pl.ds(ys, _WIN_Y), pl.ds(xs, _WIN_X)]
            m = jax.lax.dot_general(
                wyt, crop, (((1,), (0,)), ((), ())),
                precision=jax.lax.Precision.DEFAULT,
                preferred_element_type=jnp.float32)
            o = jax.lax.dot_general(
                m.astype(jnp.bfloat16), wx, (((1,), (0,)), ((), ())),
                precision=jax.lax.Precision.DEFAULT,
                preferred_element_type=jnp.float32)
            accs[c] = accs[c] + o
    for c in range(_C):
        out_ref[c, :, :] += accs[c]

    @pl.when(b == _B - 1)
    def _finish():
        out_ref[...] = out_ref[...] * pg_ref[...]


@functools.partial(jax.jit, static_argnames=())
def kernel(gradients, patch_boxes, transform_decisions, patch_grads):
    del transform_decisions
    g = jnp.transpose(gradients.astype(jnp.bfloat16), (0, 3, 1, 2))
    pg = jnp.transpose(patch_grads, (2, 0, 1))
    out = pl.pallas_call(
        _body,
        grid=(_B,),
        in_specs=[
            pl.BlockSpec((1, _C, _H, _W), lambda b: (b, 0, 0, 0)),
            pl.BlockSpec(memory_space=pltpu.SMEM),
            pl.BlockSpec((_C, _OUT, _OUT), lambda b: (0, 0, 0)),
        ],
        out_specs=pl.BlockSpec((_C, _OUT, _OUT), lambda b: (0, 0, 0)),
        out_shape=jax.ShapeDtypeStruct((_C, _OUT, _OUT), jnp.float32),
    )(g, patch_boxes, pg)
    return jnp.transpose(out, (1, 2, 0))


# speedup vs baseline: 2.8933x; 2.8933x over previous
import functools

import jax
import jax.numpy as jnp
from jax.experimental import pallas as pl
from jax.experimental.pallas import tpu as pltpu

_B, _H, _W, _C = 16, 512, 512, 3
_P = 8
_OUT = 64
_BG = 4
_WIN_Y = 136
_WIN_X = 256


def _weights_un(length, off, win, *, transposed):
    lf = length.astype(jnp.float32)
    inv_scale = lf * (1.0 / _OUT)
    ks = jnp.maximum(inv_scale, 1.0)
    if transposed:
        shape = (_OUT, win)
        s_dim, i_dim = 0, 1
    else:
        shape = (win, _OUT)
        s_dim, i_dim = 1, 0
    s = jax.lax.broadcasted_iota(jnp.int32, shape, s_dim)
    i = jax.lax.broadcasted_iota(jnp.int32, shape, i_dim)
    sample = (s.astype(jnp.float32) + 0.5) * inv_scale - 0.5
    r = i.astype(jnp.float32) - off.astype(jnp.float32)
    x = jnp.abs(sample - r) * (1.0 / ks)
    w = jnp.maximum(0.0, 1.0 - x)
    return w * ((r >= 0.0) & (r < lf)).astype(jnp.float32)


def _dot(a, b):
    return jax.lax.dot_general(
        a, b, (((1,), (0,)), ((), ())),
        precision=jax.lax.Precision.DEFAULT,
        preferred_element_type=jnp.float32)


def _body(g_ref, boxes_ref, pg_ref, out_ref):
    gb = pl.program_id(0)

    @pl.when(gb == 0)
    def _init():
        out_ref[...] = jnp.zeros_like(out_ref)

    ones_y = jnp.ones((_WIN_Y, 1), jnp.float32)
    ones_x = jnp.ones((1, _WIN_X), jnp.float32)
    accs = [jnp.zeros((_OUT, _OUT), jnp.float32) for _ in range(_C)]
    for bi in range(_BG):
        b = gb * _BG + bi
        for p in range(_P):
            ymin = boxes_ref[b, p, 0]
            xmin = boxes_ref[b, p, 1]
            ph = boxes_ref[b, p, 2]
            pw = boxes_ref[b, p, 3]
            ys = (jnp.minimum(ymin, _H - _WIN_Y + 5) // 8) * 8
            xs = (jnp.minimum(xmin, _W - _WIN_X + 1) // 128) * 128
            wyt = _weights_un(ph, ymin - ys, _WIN_Y, transposed=True)
            wx = _weights_un(pw, xmin - xs, _WIN_X, transposed=False)
            rty = 1.0 / _dot(wyt, ones_y)
            rtx = 1.0 / _dot(ones_x, wx)
            scale = rty * rtx
            ms = []
            for c in range(_C):
                crop = g_ref[bi, c, pl.ds(ys, _WIN_Y), pl.ds(xs, _WIN_X)]
                ms.append(_dot(wyt, crop))
            m_all = jnp.concatenate(ms, axis=0)
            o_all = _dot(m_all, wx)
            for c in range(_C):
                o = o_all[c * _OUT:(c + 1) * _OUT, :]
                accs[c] = accs[c] + o * scale
    for c in range(_C):
        out_ref[c, :, :] += accs[c]

    @pl.when(gb == (_B // _BG) - 1)
    def _finish():
        out_ref[...] = out_ref[...] * pg_ref[...]


@functools.partial(jax.jit, static_argnames=())
def kernel(gradients, patch_boxes, transform_decisions, patch_grads):
    del transform_decisions
    g = jnp.transpose(gradients, (0, 3, 1, 2))
    pg = jnp.transpose(patch_grads, (2, 0, 1))
    out = pl.pallas_call(
        _body,
        grid=(_B // _BG,),
        in_specs=[
            pl.BlockSpec((_BG, _C, _H, _W), lambda i: (i, 0, 0, 0)),
            pl.BlockSpec(memory_space=pltpu.SMEM),
            pl.BlockSpec((_C, _OUT, _OUT), lambda i: (0, 0, 0)),
        ],
        out_specs=pl.BlockSpec((_C, _OUT, _OUT), lambda i: (0, 0, 0)),
        out_shape=jax.ShapeDtypeStruct((_C, _OUT, _OUT), jnp.float32),
    )(g, patch_boxes, pg)
    return jnp.transpose(out, (1, 2, 0))


# speedup vs baseline: 3.1722x; 1.0964x over previous
import functools

import jax
import jax.numpy as jnp
from jax.experimental import pallas as pl
from jax.experimental.pallas import tpu as pltpu

_B, _H, _W, _C = 16, 512, 512, 3
_P = 8
_OUT = 64
_BG = 4
_WIN_Y = 136
_WIN_X = 256


def _weights_un(length, off, win, *, transposed):
    lf = length.astype(jnp.float32)
    inv_scale = lf * (1.0 / _OUT)
    ks = jnp.maximum(inv_scale, 1.0)
    if transposed:
        shape = (_OUT, win)
        s_dim, i_dim = 0, 1
    else:
        shape = (win, _OUT)
        s_dim, i_dim = 1, 0
    s = jax.lax.broadcasted_iota(jnp.int32, shape, s_dim)
    i = jax.lax.broadcasted_iota(jnp.int32, shape, i_dim)
    sample = (s.astype(jnp.float32) + 0.5) * inv_scale - 0.5
    r = i.astype(jnp.float32) - off.astype(jnp.float32)
    x = jnp.abs(sample - r) * (1.0 / ks)
    w = jnp.maximum(0.0, 1.0 - x)
    return w * ((r >= 0.0) & (r < lf)).astype(jnp.float32)


def _dot(a, b):
    return jax.lax.dot_general(
        a, b, (((1,), (0,)), ((), ())),
        precision=jax.lax.Precision.DEFAULT,
        preferred_element_type=jnp.float32)


def _body(g_ref, boxes_ref, pg_ref, out_ref):
    gb = pl.program_id(0)

    @pl.when(gb == 0)
    def _init():
        out_ref[...] = jnp.zeros_like(out_ref)

    ones_x = jnp.ones((1, _WIN_X), jnp.float32)
    accs = [jnp.zeros((_OUT, _OUT), jnp.float32) for _ in range(_C)]
    for bi in range(_BG):
        b = gb * _BG + bi
        for p in range(_P):
            ymin = boxes_ref[b, p, 0]
            xmin = boxes_ref[b, p, 1]
            ph = boxes_ref[b, p, 2]
            pw = boxes_ref[b, p, 3]
            ys = (jnp.minimum(ymin, _H - _WIN_Y + 5) // 8) * 8
            xs = (jnp.minimum(xmin, _W - _WIN_X + 1) // 128) * 128
            wyt = _weights_un(ph, ymin - ys, _WIN_Y, transposed=True)
            wx = _weights_un(pw, xmin - xs, _WIN_X, transposed=False)
            rty = 1.0 / jnp.sum(wyt, axis=1, keepdims=True)
            rtx = 1.0 / _dot(ones_x, wx)
            scale = rty * rtx
            ms = []
            for c in range(_C):
                crop = g_ref[bi, c, pl.ds(ys, _WIN_Y), pl.ds(xs, _WIN_X)]
                ms.append(_dot(wyt, crop))
            m_all = jnp.concatenate(ms, axis=0)
            o_all = _dot(m_all, wx)
            for c in range(_C):
                o = o_all[c * _OUT:(c + 1) * _OUT, :]
                accs[c] = accs[c] + o * scale
    for c in range(_C):
        out_ref[c, :, :] += accs[c]

    @pl.when(gb == (_B // _BG) - 1)
    def _finish():
        out_ref[...] = out_ref[...] * pg_ref[...]


@functools.partial(jax.jit, static_argnames=())
def kernel(gradients, patch_boxes, transform_decisions, patch_grads):
    del transform_decisions
    g = jnp.transpose(gradients, (0, 3, 1, 2))
    pg = jnp.transpose(patch_grads, (2, 0, 1))
    out = pl.pallas_call(
        _body,
        grid=(_B // _BG,),
        in_specs=[
            pl.BlockSpec((_BG, _C, _H, _W), lambda i: (i, 0, 0, 0)),
            pl.BlockSpec(memory_space=pltpu.SMEM),
            pl.BlockSpec((_C, _OUT, _OUT), lambda i: (0, 0, 0)),
        ],
        out_specs=pl.BlockSpec((_C, _OUT, _OUT), lambda i: (0, 0, 0)),
        out_shape=jax.ShapeDtypeStruct((_C, _OUT, _OUT), jnp.float32),
    )(g, patch_boxes, pg)
    return jnp.transpose(out, (1, 2, 0))
